# baseline (speedup 1.0000x reference)
import functools

import jax
import jax.numpy as jnp
from jax import lax
from jax.experimental import pallas as pl
from jax.experimental.pallas import tpu as pltpu
from jax.experimental.pallas import tpu_sc as plsc

VOCAB = 1000000
EMBED_DIM = 32
BATCH = 16384
HIST = 50

NBUF = 8
LOOKAHEAD = 6


def _make_kernel(n_workers: int, nc: int):
    nstep = BATCH // n_workers
    mesh = plsc.VectorSubcoreMesh(core_axis_name="c", subcore_axis_name="s")

    @functools.partial(
        pl.kernel,
        out_type=jax.ShapeDtypeStruct((BATCH, HIST, EMBED_DIM), jnp.float32),
        mesh=mesh,
        scratch_types=[
            pltpu.VMEM((nstep, HIST), jnp.int32),
            pltpu.VMEM((NBUF, HIST, EMBED_DIM), jnp.float32),
            pltpu.SemaphoreType.DMA((NBUF,)),
            pltpu.SemaphoreType.DMA((NBUF,)),
        ],
        compiler_params=pltpu.CompilerParams(use_tc_tiling_on_sc=False),
    )
    def k(idx_hbm, table_hbm, out_hbm, idx_v, rows_v, gsem, wsem):
        wid = lax.axis_index("s") * nc + lax.axis_index("c")
        base = wid * nstep
        pltpu.sync_copy(idx_hbm.at[pl.ds(base, nstep)], idx_v)

        for b in range(LOOKAHEAD):
            pltpu.async_copy(table_hbm.at[idx_v.at[b]], rows_v.at[b],
                             gsem.at[b])

        def block(j0, carry):
            for b in range(NBUF):
                j = j0 + b
                jn = j + LOOKAHEAD
                bn = (b + LOOKAHEAD) % NBUF

                @pl.when(jn < nstep)
                def _():
                    @pl.when(jn >= NBUF)
                    def _():
                        pltpu.make_async_copy(
                            rows_v.at[bn], out_hbm.at[base],
                            wsem.at[bn]).wait()
                    pltpu.async_copy(table_hbm.at[idx_v.at[jn]],
                                     rows_v.at[bn], gsem.at[bn])

                pltpu.make_async_copy(
                    table_hbm.at[idx_v.at[j]], rows_v.at[b],
                    gsem.at[b]).wait()
                pltpu.async_copy(rows_v.at[b], out_hbm.at[base + j],
                                 wsem.at[b])
            return carry

        lax.fori_loop(0, nstep // NBUF, lambda i, c: block(i * NBUF, c), 0)

        for b in range(NBUF):
            pltpu.make_async_copy(rows_v.at[b], out_hbm.at[base],
                                  wsem.at[b]).wait()

    return k


def kernel(input, table):
    info = plsc.get_sparse_core_info()
    n_workers = info.num_cores * info.num_subcores
    r = lax.optimization_barrier(table.reshape(VOCAB // 4, 4 * EMBED_DIM))
    return _make_kernel(n_workers, info.num_cores)(
        input.astype(jnp.int32), r.reshape(VOCAB, EMBED_DIM))


# speedup vs baseline: 1.0151x; 1.0151x over previous
import functools

import jax
import jax.numpy as jnp
from jax import lax
from jax.experimental import pallas as pl
from jax.experimental.pallas import tpu as pltpu
from jax.experimental.pallas import tpu_sc as plsc

VOCAB = 1000000
EMBED_DIM = 32
BATCH = 16384
HIST = 50

NBUF = 8
LOOKAHEAD = 6


def _make_kernel(n_workers: int, nc: int):
    nstep = BATCH // n_workers
    mesh = plsc.VectorSubcoreMesh(core_axis_name="c", subcore_axis_name="s")

    @functools.partial(
        pl.kernel,
        out_type=jax.ShapeDtypeStruct((BATCH, HIST, EMBED_DIM), jnp.float32),
        mesh=mesh,
        scratch_types=[
            pltpu.VMEM((nstep, HIST), jnp.int32),
            pltpu.VMEM((NBUF, HIST, EMBED_DIM), jnp.float32),
            pltpu.SemaphoreType.DMA((NBUF,)),
            pltpu.SemaphoreType.DMA((NBUF,)),
        ],
        compiler_params=pltpu.CompilerParams(use_tc_tiling_on_sc=False),
    )
    def k(idx_hbm, table_hbm, out_hbm, idx_v, rows_v, gsem, wsem):
        wid = lax.axis_index("s") * nc + lax.axis_index("c")
        base = wid * nstep
        pltpu.sync_copy(idx_hbm.at[pl.ds(base, nstep)], idx_v)

        for b in range(LOOKAHEAD):
            pltpu.async_copy(table_hbm.at[idx_v.at[b]], rows_v.at[b],
                             gsem.at[b])

        def block(j0, carry):
            for b in range(NBUF):
                j = j0 + b
                jn = j + LOOKAHEAD
                bn = (b + LOOKAHEAD) % NBUF

                @pl.when(jn < nstep)
                def _():
                    @pl.when(jn >= NBUF)
                    def _():
                        pltpu.make_async_copy(
                            rows_v.at[bn], out_hbm.at[base],
                            wsem.at[bn]).wait()
                    pltpu.async_copy(table_hbm.at[idx_v.at[jn]],
                                     rows_v.at[bn], gsem.at[bn])

                pltpu.make_async_copy(
                    table_hbm.at[idx_v.at[j]], rows_v.at[b],
                    gsem.at[b]).wait()
                pltpu.async_copy(rows_v.at[b], out_hbm.at[base + j],
                                 wsem.at[b])
            return carry

        lax.fori_loop(0, nstep // NBUF, lambda i, c: block(i * NBUF, c), 0)

        for b in range(NBUF):
            pltpu.make_async_copy(rows_v.at[b], out_hbm.at[base],
                                  wsem.at[b]).wait()

    return k


def kernel(input, table):
    info = plsc.get_sparse_core_info()
    n_workers = info.num_cores * info.num_subcores
    tblp = jnp.pad(table, ((0, 0), (0, 128 - EMBED_DIM)))
    return _make_kernel(n_workers, info.num_cores)(
        input.astype(jnp.int32) * 4,
        tblp.reshape(4 * VOCAB, EMBED_DIM))
